# Initial kernel scaffold; baseline (speedup 1.0000x reference)
import functools

import jax
import jax.numpy as jnp
from jax import lax
from jax.experimental import pallas as pl
from jax.experimental.pallas import tpu as pltpu
from jax.experimental.pallas import tpu_sc as plsc

_N = 10000
_E = 320000
_D = 128
_NC = 2
_NS = 16
_NW = _NC * _NS
_L = 16

_CHUNK = 128
_NCHUNKS = _E // _CHUNK
_BASE_CHUNKS = _NCHUNKS // _NW
_EXTRA = _NCHUNKS - _BASE_CHUNKS * _NW
_STRIPE = _N // _NS


def _mm_body(x_ref, w_ref, o_ref):
    o_ref[...] = jnp.dot(x_ref[...], w_ref[...],
                         preferred_element_type=jnp.float32)


def _matmul(x, w):
    return pl.pallas_call(
        _mm_body,
        grid=(5,),
        in_specs=[
            pl.BlockSpec((2000, _D), lambda i: (i, 0)),
            pl.BlockSpec((_D, _D), lambda i: (0, 0)),
        ],
        out_specs=pl.BlockSpec((2000, _D), lambda i: (i, 0)),
        out_shape=jax.ShapeDtypeStruct((_N, _D), jnp.float32),
    )(x, w)


_mesh = plsc.VectorSubcoreMesh(core_axis_name="c", subcore_axis_name="s")


@functools.partial(
    pl.kernel,
    out_type=jax.ShapeDtypeStruct((_NC, _N, _D), jnp.float32),
    mesh=_mesh,
    scratch_types=[
        pltpu.VMEM((_CHUNK,), jnp.int32),
        pltpu.VMEM((_CHUNK,), jnp.int32),
        pltpu.VMEM((_CHUNK,), jnp.float32),
        pltpu.VMEM((_CHUNK, _D), jnp.float32),
        pltpu.VMEM_SHARED((_N, _D), jnp.float32),
        pltpu.SemaphoreType.DMA,
    ],
)
def _spmm(src_hbm, dst_hbm, ew_hbm, sup_hbm, out_hbm,
          src_v, dst_v, w_v, rows_v, acc, sem):
    c = lax.axis_index("c")
    s = lax.axis_index("s")
    wid = s * _NC + c

    def _zero_row(i, carry):
        for j in range(_D // _L):
            rows_v[i, pl.ds(j * _L, _L)] = jnp.zeros((_L,), jnp.float32)
        return carry
    lax.fori_loop(0, _CHUNK, _zero_row, 0)

    stripe = s * _STRIPE
    for k in range(_STRIPE // _CHUNK):
        pltpu.sync_copy(rows_v, acc.at[pl.ds(stripe + k * _CHUNK, _CHUNK)])
    if _STRIPE % _CHUNK:
        pltpu.sync_copy(rows_v,
                        acc.at[pl.ds(stripe + _STRIPE - _CHUNK, _CHUNK)])
    plsc.subcore_barrier()

    nchunks = _BASE_CHUNKS + jnp.where(wid < _EXTRA, 1, 0)

    def _edge_chunk(i, carry):
        base = (i * _NW + wid) * _CHUNK
        pltpu.sync_copy(src_hbm.at[pl.ds(base, _CHUNK)], src_v)
        pltpu.sync_copy(dst_hbm.at[pl.ds(base, _CHUNK)], dst_v)
        pltpu.sync_copy(ew_hbm.at[pl.ds(base, _CHUNK)], w_v)
        pltpu.async_copy(sup_hbm.at[src_v], rows_v, sem).wait()

        def _scale16(g, carry2):
            wvec = w_v[pl.ds(g * _L, _L)]
            for l in range(_L):
                wl = jnp.take(wvec, jnp.full((_L,), l, jnp.int32),
                              mode="promise_in_bounds")
                r = g * _L + l
                for j in range(_D // _L):
                    sl = pl.ds(j * _L, _L)
                    rows_v[r, sl] = rows_v[r, sl] * wl
            return carry2
        lax.fori_loop(0, _CHUNK // _L, _scale16, 0)

        pltpu.sync_copy(rows_v, acc.at[dst_v], add=True)
        return carry
    lax.fori_loop(0, nchunks, _edge_chunk, 0)

    plsc.subcore_barrier()
    pltpu.sync_copy(acc.at[pl.ds(stripe, _STRIPE)],
                    out_hbm.at[c, pl.ds(stripe, _STRIPE)])


def _comb_body(p_ref, b_ref, o_ref):
    o_ref[...] = p_ref[0] + p_ref[1] + b_ref[...]


def _combine(partials, bias2d):
    return pl.pallas_call(
        _comb_body,
        grid=(5,),
        in_specs=[
            pl.BlockSpec((_NC, 2000, _D), lambda i: (0, i, 0)),
            pl.BlockSpec((1, _D), lambda i: (0, 0)),
        ],
        out_specs=pl.BlockSpec((2000, _D), lambda i: (i, 0)),
        out_shape=jax.ShapeDtypeStruct((_N, _D), jnp.float32),
    )(partials, bias2d)


def kernel(x, edge_index, edge_weight, weight, bias):
    support = _matmul(x, weight)
    partials = _spmm(edge_index[0], edge_index[1], edge_weight, support)
    return _combine(partials, bias.reshape(1, _D))


# speedup vs baseline: 5.4553x; 5.4553x over previous
import functools

import jax
import jax.numpy as jnp
from jax import lax
from jax.experimental import pallas as pl
from jax.experimental.pallas import tpu as pltpu
from jax.experimental.pallas import tpu_sc as plsc

_N = 10000
_E = 320000
_D = 128
_NC = 2
_NS = 16
_NW = _NC * _NS
_L = 16

_CHUNK = 128
_NCHUNKS = _E // _CHUNK
_BASE_CHUNKS = _NCHUNKS // _NW
_EXTRA = _NCHUNKS - _BASE_CHUNKS * _NW
_STRIPE = 624


def _mm_body(x_ref, w_ref, o_ref):
    o_ref[...] = jnp.dot(x_ref[...], w_ref[...],
                         preferred_element_type=jnp.float32)


def _matmul(x, w):
    return pl.pallas_call(
        _mm_body,
        grid=(5,),
        in_specs=[
            pl.BlockSpec((2000, _D), lambda i: (i, 0)),
            pl.BlockSpec((_D, _D), lambda i: (0, 0)),
        ],
        out_specs=pl.BlockSpec((2000, _D), lambda i: (i, 0)),
        out_shape=jax.ShapeDtypeStruct((_N, _D), jnp.float32),
    )(x, w)


_mesh = plsc.VectorSubcoreMesh(core_axis_name="c", subcore_axis_name="s")


@functools.partial(
    pl.kernel,
    out_type=jax.ShapeDtypeStruct((_NC, _N, _D), jnp.float32),
    mesh=_mesh,
    scratch_types=[
        pltpu.VMEM((_CHUNK,), jnp.int32),
        pltpu.VMEM((_CHUNK,), jnp.int32),
        pltpu.VMEM((_CHUNK,), jnp.float32),
        pltpu.VMEM((_CHUNK, _D), jnp.float32),
        pltpu.VMEM_SHARED((_N, _D), jnp.float32),
        pltpu.SemaphoreType.DMA,
    ],
)
def _spmm(src_hbm, dst_hbm, ew_hbm, sup_hbm, out_hbm,
          src_v, dst_v, w_v, rows_v, acc, sem):
    c = lax.axis_index("c")
    s = lax.axis_index("s")
    wid = s * _NC + c

    def _zero_row(i, carry):
        for j in range(_D // _L):
            rows_v[i, pl.ds(j * _L, _L)] = jnp.zeros((_L,), jnp.float32)
        return carry
    lax.fori_loop(0, _CHUNK, _zero_row, 0)

    stripe = s * _STRIPE
    for off in (0, 128, 256, 384, 496):
        pltpu.sync_copy(rows_v, acc.at[pl.ds(stripe + off, _CHUNK)])
    @pl.when(s == _NS - 1)
    def _zero_tail():
        pltpu.sync_copy(rows_v, acc.at[pl.ds(_N - _CHUNK, _CHUNK)])
    plsc.subcore_barrier()

    nchunks = _BASE_CHUNKS + jnp.where(wid < _EXTRA, 1, 0)

    def _edge_chunk(i, carry):
        base = (i * _NW + wid) * _CHUNK
        pltpu.sync_copy(src_hbm.at[pl.ds(base, _CHUNK)], src_v)
        pltpu.sync_copy(dst_hbm.at[pl.ds(base, _CHUNK)], dst_v)
        pltpu.sync_copy(ew_hbm.at[pl.ds(base, _CHUNK)], w_v)
        pltpu.async_copy(sup_hbm.at[src_v], rows_v, sem).wait()

        def _scale16(g, carry2):
            wvec = w_v[pl.ds(g * _L, _L)]
            for l in range(_L):
                wl = wvec.at[jnp.full((_L,), l, jnp.int32)].get(
                    mode="promise_in_bounds")
                r = g * _L + l
                for j in range(_D // _L):
                    sl = pl.ds(j * _L, _L)
                    rows_v[r, sl] = rows_v[r, sl] * wl
            return carry2
        lax.fori_loop(0, _CHUNK // _L, _scale16, 0)

        pltpu.sync_copy(rows_v, acc.at[dst_v], add=True)
        return carry
    lax.fori_loop(0, nchunks, _edge_chunk, 0)

    plsc.subcore_barrier()
    for off in (0, 128, 256, 384, 496):
        pltpu.sync_copy(acc.at[pl.ds(stripe + off, _CHUNK)],
                        out_hbm.at[c, pl.ds(stripe + off, _CHUNK)])

    @pl.when(s == _NS - 1)
    def _write_tail():
        pltpu.sync_copy(acc.at[pl.ds(_N - _CHUNK, _CHUNK)],
                        out_hbm.at[c, pl.ds(_N - _CHUNK, _CHUNK)])


def _comb_body(p_ref, b_ref, o_ref):
    o_ref[...] = p_ref[0] + p_ref[1] + b_ref[...]


def _combine(partials, bias2d):
    return pl.pallas_call(
        _comb_body,
        grid=(5,),
        in_specs=[
            pl.BlockSpec((_NC, 2000, _D), lambda i: (0, i, 0)),
            pl.BlockSpec((1, _D), lambda i: (0, 0)),
        ],
        out_specs=pl.BlockSpec((2000, _D), lambda i: (i, 0)),
        out_shape=jax.ShapeDtypeStruct((_N, _D), jnp.float32),
    )(partials, bias2d)


def kernel(x, edge_index, edge_weight, weight, bias):
    support = _matmul(x, weight)
    partials = _spmm(edge_index[0], edge_index[1], edge_weight, support)
    return _combine(partials, bias.reshape(1, _D))
